# baseline (speedup 1.0000x reference)
import jax
import jax.numpy as jnp
from jax import lax
from jax.experimental import pallas as pl
from jax.experimental.pallas import tpu as pltpu

_F32 = jnp.float32
_BF16 = jnp.bfloat16


def _layer_norm(v, g, be):
    n = v.shape[1]
    s = jnp.sum(v, axis=1, keepdims=True)
    ss = jnp.sum(v * v, axis=1, keepdims=True)
    m = s * (1.0 / n)
    var = ss * (1.0 / n) - m * m
    k = lax.rsqrt(var + 1e-5)
    return (v - m) * k * g + be


def _shift_down(y):
    return jnp.concatenate([jnp.zeros((1, y.shape[1]), y.dtype), y[:-1, :]], axis=0)


def _shift_up(y):
    return jnp.concatenate([y[1:, :], jnp.zeros((1, y.shape[1]), y.dtype)], axis=0)


def _conv3(hb, w):
    y0 = jnp.dot(hb, w[0], preferred_element_type=_F32)
    y1 = jnp.dot(hb, w[1], preferred_element_type=_F32)
    y2 = jnp.dot(hb, w[2], preferred_element_type=_F32)
    return y1 + _shift_down(y0) + _shift_up(y2)


def _vp(hb, w1, b1, g1, be1, w2, b2, g2, be2, lw_col, lb, mask_row):
    c = _conv3(hb, w1) + b1
    c = jnp.maximum(c, 0.0)
    c = _layer_norm(c, g1, be1)
    c2 = _conv3(c.astype(_BF16), w2) + b2
    c2 = jnp.maximum(c2, 0.0)
    c2 = _layer_norm(c2, g2, be2)
    pred = jnp.dot(c2.astype(_BF16), lw_col, preferred_element_type=_F32) + lb[0, 0]
    pred_row = pred.reshape(1, pred.shape[0])
    return jnp.where(mask_row != 0, 0.0, pred_row)


def _body(x_ref, dur_ref, smask_ref, mmask_ref, pt_ref, et_ref, ml_ref,
          pbh_ref, pbl_ref, ebh_ref, ebl_ref, ptab_ref, etab_ref,
          dw1, db1, dg1, dbe1, dw2, db2, dg2, dbe2, dlw, dlb,
          pw1, pb1, pg1, pbe1, pw2, pb2, pg2, pbe2, plw, plb,
          ew1, eb1, eg1, ebe1, ew2, eb2, eg2, ebe2, elw, elb,
          out_ref, logd_ref, ppred_ref, epred_ref, mellen_ref):
    S = x_ref.shape[1]
    T = out_ref.shape[1]

    xb = x_ref[0].astype(_BF16)

    logd_ref[0] = _vp(xb, dw1[...], db1[...], dg1[...], dbe1[...],
                      dw2[...], db2[...], dg2[...], dbe2[...],
                      dlw[...], dlb[...], smask_ref[0])

    ii = lax.broadcasted_iota(jnp.int32, (S, S), 0)
    jj = lax.broadcasted_iota(jnp.int32, (S, S), 1)
    tri = (ii <= jj).astype(_BF16)
    durb = dur_ref[0].astype(_BF16)
    cum = jnp.dot(durb, tri, preferred_element_type=_F32)
    cumsh = jnp.concatenate([jnp.zeros((1, 1), _F32), cum[:, :-1]], axis=1)
    mlen_f = jnp.minimum(cum[:, S - 1:S], ml_ref[0, 0].astype(_F32))
    pos = lax.broadcasted_iota(jnp.int32, (T, 1), 0).astype(_F32)
    valid = pos < mlen_f
    oh = jnp.logical_and(cum > pos, cumsh <= pos)
    oh = jnp.logical_and(oh, valid).astype(_F32)
    x_exp = jnp.dot(oh, x_ref[0], preferred_element_type=_F32)

    xeb = x_exp.astype(_BF16)
    mmask = mmask_ref[0]
    ppred_ref[0] = _vp(xeb, pw1[...], pb1[...], pg1[...], pbe1[...],
                       pw2[...], pb2[...], pg2[...], pbe2[...],
                       plw[...], plb[...], mmask)
    epred_ref[0] = _vp(xeb, ew1[...], eb1[...], eg1[...], ebe1[...],
                       ew2[...], eb2[...], eg2[...], ebe2[...],
                       elw[...], elb[...], mmask)

    ptc = pt_ref[0].reshape(T, 1)
    etc = et_ref[0].reshape(T, 1)
    ohp = ((pbh_ref[...] >= ptc) & (pbl_ref[...] < ptc)).astype(_F32)
    ohe = ((ebh_ref[...] >= etc) & (ebl_ref[...] < etc)).astype(_F32)
    pemb = jnp.dot(ohp, ptab_ref[...], preferred_element_type=_F32)
    eemb = jnp.dot(ohe, etab_ref[...], preferred_element_type=_F32)
    out_ref[0] = x_exp + pemb + eemb

    mel_i = jnp.minimum(cum[:, S - 1:S].astype(jnp.int32), ml_ref[0, 0])
    mellen_ref[0] = jnp.broadcast_to(mel_i, (1, 128))


def kernel(x, src_mask, mel_mask, duration_target, pitch_target, energy_target, max_len, pitch_bins, energy_bins, pitch_table, energy_table, dp_w1, dp_b1, dp_g1, dp_be1, dp_w2, dp_b2, dp_g2, dp_be2, dp_lw, dp_lb, pp_w1, pp_b1, pp_g1, pp_be1, pp_w2, pp_b2, pp_g2, pp_be2, pp_lw, pp_lb, ep_w1, ep_b1, ep_g1, ep_be1, ep_w2, ep_b2, ep_g2, ep_be2, ep_lw, ep_lb):
    B, S, H = x.shape
    T = mel_mask.shape[1]
    F = dp_b1.shape[0]
    NB = pitch_table.shape[0]

    smask = src_mask.reshape(B, 1, S).astype(jnp.int32)
    mmask = mel_mask.reshape(B, 1, T).astype(jnp.int32)
    dur = duration_target.reshape(B, 1, S).astype(jnp.int32)
    pt = pitch_target.reshape(B, 1, T)
    et = energy_target.reshape(B, 1, T)
    ml = jnp.asarray(max_len, jnp.int32).reshape(1, 1)

    inf = jnp.full((1,), jnp.inf, _F32)
    pbh = jnp.concatenate([pitch_bins, inf]).reshape(1, NB)
    pbl = jnp.concatenate([-inf, pitch_bins]).reshape(1, NB)
    ebh = jnp.concatenate([energy_bins, inf]).reshape(1, NB)
    ebl = jnp.concatenate([-inf, energy_bins]).reshape(1, NB)

    def vp_args(w1, b1, g1, be1, w2, b2, g2, be2, lw, lb):
        return (w1.astype(_BF16), b1.reshape(1, F),
                g1.reshape(1, F), be1.reshape(1, F),
                w2.astype(_BF16), b2.reshape(1, F),
                g2.reshape(1, F), be2.reshape(1, F), lw.astype(_BF16),
                lb.reshape(1, 1))

    dp = vp_args(dp_w1, dp_b1, dp_g1, dp_be1, dp_w2, dp_b2, dp_g2, dp_be2, dp_lw, dp_lb)
    pp = vp_args(pp_w1, pp_b1, pp_g1, pp_be1, pp_w2, pp_b2, pp_g2, pp_be2, pp_lw, pp_lb)
    ep = vp_args(ep_w1, ep_b1, ep_g1, ep_be1, ep_w2, ep_b2, ep_g2, ep_be2, ep_lw, ep_lb)

    def full(a):
        return pl.BlockSpec(a.shape, lambda b: (0,) * a.ndim)

    in_specs = [
        pl.BlockSpec((1, S, H), lambda b: (b, 0, 0)),
        pl.BlockSpec((1, 1, S), lambda b: (b, 0, 0)),
        pl.BlockSpec((1, 1, S), lambda b: (b, 0, 0)),
        pl.BlockSpec((1, 1, T), lambda b: (b, 0, 0)),
        pl.BlockSpec((1, 1, T), lambda b: (b, 0, 0)),
        pl.BlockSpec((1, 1, T), lambda b: (b, 0, 0)),
        pl.BlockSpec(memory_space=pltpu.SMEM),
        full(pbh), full(pbl), full(ebh), full(ebl),
        full(pitch_table), full(energy_table),
    ]
    for grp in (dp, pp, ep):
        in_specs.extend(full(a) for a in grp)

    out_shapes = (
        jax.ShapeDtypeStruct((B, T, H), _F32),
        jax.ShapeDtypeStruct((B, 1, S), _F32),
        jax.ShapeDtypeStruct((B, 1, T), _F32),
        jax.ShapeDtypeStruct((B, 1, T), _F32),
        jax.ShapeDtypeStruct((B, 1, 128), jnp.int32),
    )
    out_specs = (
        pl.BlockSpec((1, T, H), lambda b: (b, 0, 0)),
        pl.BlockSpec((1, 1, S), lambda b: (b, 0, 0)),
        pl.BlockSpec((1, 1, T), lambda b: (b, 0, 0)),
        pl.BlockSpec((1, 1, T), lambda b: (b, 0, 0)),
        pl.BlockSpec((1, 1, 128), lambda b: (b, 0, 0)),
    )

    out, logd, ppred, epred, mellen = pl.pallas_call(
        _body,
        grid=(B,),
        in_specs=in_specs,
        out_specs=out_specs,
        out_shape=out_shapes,
    )(x, dur, smask, mmask, pt, et, ml, pbh, pbl, ebh, ebl,
      pitch_table, energy_table, *dp, *pp, *ep)

    return (out, logd.reshape(B, S), ppred.reshape(B, T), epred.reshape(B, T),
            mellen[:, 0, 0], mel_mask)


# speedup vs baseline: 1.0473x; 1.0473x over previous
import jax
import jax.numpy as jnp
from jax import lax
from jax.experimental import pallas as pl
from jax.experimental.pallas import tpu as pltpu

_F32 = jnp.float32
_BF16 = jnp.bfloat16
_G = 2


def _layer_norm(v, g, be):
    n = v.shape[1]
    s = jnp.sum(v, axis=1, keepdims=True)
    ss = jnp.sum(v * v, axis=1, keepdims=True)
    m = s * (1.0 / n)
    var = ss * (1.0 / n) - m * m
    k = lax.rsqrt(var + 1e-5)
    return (v - m) * k * g + be


def _shift_down(y):
    return jnp.concatenate([jnp.zeros((1, y.shape[1]), y.dtype), y[:-1, :]], axis=0)


def _shift_up(y):
    return jnp.concatenate([y[1:, :], jnp.zeros((1, y.shape[1]), y.dtype)], axis=0)


def _conv3(hb, w):
    y0 = jnp.dot(hb, w[0], preferred_element_type=_F32)
    y1 = jnp.dot(hb, w[1], preferred_element_type=_F32)
    y2 = jnp.dot(hb, w[2], preferred_element_type=_F32)
    return y1 + _shift_down(y0) + _shift_up(y2)


def _vp(hb, w1, b1, g1, be1, w2, b2, g2, be2, lw_col, lb, mask_col):
    c = _conv3(hb, w1) + b1
    c = jnp.maximum(c, 0.0)
    c = _layer_norm(c, g1, be1)
    c2 = _conv3(c.astype(_BF16), w2) + b2
    c2 = jnp.maximum(c2, 0.0)
    c2 = _layer_norm(c2, g2, be2)
    pred = jnp.dot(c2.astype(_BF16), lw_col, preferred_element_type=_F32) + lb[0, 0]
    return jnp.where(mask_col != 0, 0.0, pred)


def _body(x_ref, dur_ref, smask_ref, mmask_ref, pt_ref, et_ref, ml_ref,
          pbh_ref, pbl_ref, ebh_ref, ebl_ref, ptab_ref, etab_ref,
          dw1, db1, dg1, dbe1, dw2, db2, dg2, dbe2, dlw, dlb,
          pw1, pb1, pg1, pbe1, pw2, pb2, pg2, pbe2, plw, plb,
          ew1, eb1, eg1, ebe1, ew2, eb2, eg2, ebe2, elw, elb,
          out_ref, logd_ref, ppred_ref, epred_ref, mellen_ref):
    S = x_ref.shape[1]
    T = out_ref.shape[1]

    ii = lax.broadcasted_iota(jnp.int32, (S, S), 0)
    jj = lax.broadcasted_iota(jnp.int32, (S, S), 1)
    tri = (ii <= jj).astype(_BF16)
    pos = lax.broadcasted_iota(jnp.int32, (T, 1), 0).astype(_F32)

    for g in range(_G):
        xb = x_ref[g].astype(_BF16)

        logd_ref[g] = _vp(xb, dw1[...], db1[...], dg1[...], dbe1[...],
                          dw2[...], db2[...], dg2[...], dbe2[...],
                          dlw[...], dlb[...], smask_ref[g])

        durb = dur_ref[g].astype(_BF16)
        cum = jnp.dot(durb, tri, preferred_element_type=_F32)
        cumsh = jnp.concatenate([jnp.zeros((1, 1), _F32), cum[:, :-1]], axis=1)
        mlen_f = jnp.minimum(cum[:, S - 1:S], ml_ref[0, 0].astype(_F32))
        valid = pos < mlen_f
        oh = jnp.logical_and(cum > pos, cumsh <= pos)
        oh = jnp.logical_and(oh, valid).astype(_F32)
        x_exp = jnp.dot(oh, x_ref[g], preferred_element_type=_F32)

        xeb = x_exp.astype(_BF16)
        mmask = mmask_ref[g]
        ppred_ref[g] = _vp(xeb, pw1[...], pb1[...], pg1[...], pbe1[...],
                           pw2[...], pb2[...], pg2[...], pbe2[...],
                           plw[...], plb[...], mmask)
        epred_ref[g] = _vp(xeb, ew1[...], eb1[...], eg1[...], ebe1[...],
                           ew2[...], eb2[...], eg2[...], ebe2[...],
                           elw[...], elb[...], mmask)

        ptc = pt_ref[g]
        etc = et_ref[g]
        ohp = ((pbh_ref[...] >= ptc) & (pbl_ref[...] < ptc)).astype(_F32)
        ohe = ((ebh_ref[...] >= etc) & (ebl_ref[...] < etc)).astype(_F32)
        pemb = jnp.dot(ohp, ptab_ref[...], preferred_element_type=_F32)
        eemb = jnp.dot(ohe, etab_ref[...], preferred_element_type=_F32)
        out_ref[g] = x_exp + pemb + eemb

        mel_i = jnp.minimum(cum[:, S - 1:S].astype(jnp.int32), ml_ref[0, 0])
        mellen_ref[g] = jnp.broadcast_to(mel_i, (1, 128))


def kernel(x, src_mask, mel_mask, duration_target, pitch_target, energy_target, max_len, pitch_bins, energy_bins, pitch_table, energy_table, dp_w1, dp_b1, dp_g1, dp_be1, dp_w2, dp_b2, dp_g2, dp_be2, dp_lw, dp_lb, pp_w1, pp_b1, pp_g1, pp_be1, pp_w2, pp_b2, pp_g2, pp_be2, pp_lw, pp_lb, ep_w1, ep_b1, ep_g1, ep_be1, ep_w2, ep_b2, ep_g2, ep_be2, ep_lw, ep_lb):
    B, S, H = x.shape
    T = mel_mask.shape[1]
    F = dp_b1.shape[0]
    NB = pitch_table.shape[0]
    NG = B // _G

    smask = src_mask.reshape(B, S, 1).astype(jnp.int32)
    mmask = mel_mask.reshape(B, T, 1).astype(jnp.int32)
    dur = duration_target.reshape(B, 1, S).astype(jnp.int32)
    pt = pitch_target.reshape(B, T, 1)
    et = energy_target.reshape(B, T, 1)
    ml = jnp.asarray(max_len, jnp.int32).reshape(1, 1)

    inf = jnp.full((1,), jnp.inf, _F32)
    pbh = jnp.concatenate([pitch_bins, inf]).reshape(1, NB)
    pbl = jnp.concatenate([-inf, pitch_bins]).reshape(1, NB)
    ebh = jnp.concatenate([energy_bins, inf]).reshape(1, NB)
    ebl = jnp.concatenate([-inf, energy_bins]).reshape(1, NB)

    def vp_args(w1, b1, g1, be1, w2, b2, g2, be2, lw, lb):
        return (w1.astype(_BF16), b1.reshape(1, F),
                g1.reshape(1, F), be1.reshape(1, F),
                w2.astype(_BF16), b2.reshape(1, F),
                g2.reshape(1, F), be2.reshape(1, F), lw.astype(_BF16),
                lb.reshape(1, 1))

    dp = vp_args(dp_w1, dp_b1, dp_g1, dp_be1, dp_w2, dp_b2, dp_g2, dp_be2, dp_lw, dp_lb)
    pp = vp_args(pp_w1, pp_b1, pp_g1, pp_be1, pp_w2, pp_b2, pp_g2, pp_be2, pp_lw, pp_lb)
    ep = vp_args(ep_w1, ep_b1, ep_g1, ep_be1, ep_w2, ep_b2, ep_g2, ep_be2, ep_lw, ep_lb)

    def full(a):
        return pl.BlockSpec(a.shape, lambda b: (0,) * a.ndim)

    in_specs = [
        pl.BlockSpec((_G, S, H), lambda b: (b, 0, 0)),
        pl.BlockSpec((_G, 1, S), lambda b: (b, 0, 0)),
        pl.BlockSpec((_G, S, 1), lambda b: (b, 0, 0)),
        pl.BlockSpec((_G, T, 1), lambda b: (b, 0, 0)),
        pl.BlockSpec((_G, T, 1), lambda b: (b, 0, 0)),
        pl.BlockSpec((_G, T, 1), lambda b: (b, 0, 0)),
        pl.BlockSpec(memory_space=pltpu.SMEM),
        full(pbh), full(pbl), full(ebh), full(ebl),
        full(pitch_table), full(energy_table),
    ]
    for grp in (dp, pp, ep):
        in_specs.extend(full(a) for a in grp)

    out_shapes = (
        jax.ShapeDtypeStruct((B, T, H), _F32),
        jax.ShapeDtypeStruct((B, S, 1), _F32),
        jax.ShapeDtypeStruct((B, T, 1), _F32),
        jax.ShapeDtypeStruct((B, T, 1), _F32),
        jax.ShapeDtypeStruct((B, 1, 128), jnp.int32),
    )
    out_specs = (
        pl.BlockSpec((_G, T, H), lambda b: (b, 0, 0)),
        pl.BlockSpec((_G, S, 1), lambda b: (b, 0, 0)),
        pl.BlockSpec((_G, T, 1), lambda b: (b, 0, 0)),
        pl.BlockSpec((_G, T, 1), lambda b: (b, 0, 0)),
        pl.BlockSpec((_G, 1, 128), lambda b: (b, 0, 0)),
    )

    out, logd, ppred, epred, mellen = pl.pallas_call(
        _body,
        grid=(NG,),
        in_specs=in_specs,
        out_specs=out_specs,
        out_shape=out_shapes,
    )(x, dur, smask, mmask, pt, et, ml, pbh, pbl, ebh, ebl,
      pitch_table, energy_table, *dp, *pp, *ep)

    return (out, logd.reshape(B, S), ppred.reshape(B, T), epred.reshape(B, T),
            mellen[:, 0, 0], mel_mask)


# speedup vs baseline: 1.1472x; 1.0953x over previous
import jax
import jax.numpy as jnp
from jax import lax
from jax.experimental import pallas as pl
from jax.experimental.pallas import tpu as pltpu

_F32 = jnp.float32
_BF16 = jnp.bfloat16
_G = 2


def _layer_norm(v, g, be):
    n = v.shape[1]
    s = jnp.sum(v, axis=1, keepdims=True)
    ss = jnp.sum(v * v, axis=1, keepdims=True)
    m = s * (1.0 / n)
    var = ss * (1.0 / n) - m * m
    k = lax.rsqrt(var + 1e-5)
    return (v - m) * k * g + be


def _shift_down(y):
    return jnp.concatenate([jnp.zeros((1, y.shape[1]), y.dtype), y[:-1, :]], axis=0)


def _shift_up(y):
    return jnp.concatenate([y[1:, :], jnp.zeros((1, y.shape[1]), y.dtype)], axis=0)


def _conv3(hb, w):
    y0 = jnp.dot(hb, w[0], preferred_element_type=_F32)
    y1 = jnp.dot(hb, w[1], preferred_element_type=_F32)
    y2 = jnp.dot(hb, w[2], preferred_element_type=_F32)
    return y1 + _shift_down(y0) + _shift_up(y2)


def _vp(hb, w1, b1, g1, be1, w2, b2, g2, be2, lw_col, lb):
    c = _conv3(hb, w1) + b1
    c = jnp.maximum(c, 0.0)
    c = _layer_norm(c, g1, be1)
    c2 = _conv3(c.astype(_BF16), w2) + b2
    c2 = jnp.maximum(c2, 0.0)
    c2 = _layer_norm(c2, g2, be2)
    return jnp.dot(c2.astype(_BF16), lw_col.astype(_BF16),
                   preferred_element_type=_F32) + lb[0, 0]


def _body(x_ref, dur_ref, pt_ref, et_ref, ml_ref,
          pbins_ref, ebins_ref, ptab_ref, etab_ref,
          dw1, db1, dg1, dbe1, dw2, db2, dg2, dbe2, dlw, dlb,
          pw1, pb1, pg1, pbe1, pw2, pb2, pg2, pbe2, plw, plb,
          ew1, eb1, eg1, ebe1, ew2, eb2, eg2, ebe2, elw, elb,
          out_ref, logd_ref, ppred_ref, epred_ref, mellen_ref,
          dw1s, dw2s, pw1s, pw2s, ew1s, ew2s):
    S = x_ref.shape[1]
    T = out_ref.shape[1]
    NB = ptab_ref.shape[0]

    @pl.when(pl.program_id(0) == 0)
    def _cast_weights():
        dw1s[...] = dw1[...].astype(_BF16)
        dw2s[...] = dw2[...].astype(_BF16)
        pw1s[...] = pw1[...].astype(_BF16)
        pw2s[...] = pw2[...].astype(_BF16)
        ew1s[...] = ew1[...].astype(_BF16)
        ew2s[...] = ew2[...].astype(_BF16)

    ii = lax.broadcasted_iota(jnp.int32, (S, S), 0)
    jj = lax.broadcasted_iota(jnp.int32, (S, S), 1)
    tri = (ii <= jj).astype(_BF16)
    pos = lax.broadcasted_iota(jnp.int32, (T, 1), 0).astype(_F32)

    big = jnp.full((1, 1), jnp.inf, _F32)
    pbh = jnp.concatenate([pbins_ref[...], big], axis=1)
    pbl = jnp.concatenate([-big, pbins_ref[...]], axis=1)
    ebh = jnp.concatenate([ebins_ref[...], big], axis=1)
    ebl = jnp.concatenate([-big, ebins_ref[...]], axis=1)

    for g in range(_G):
        xb = x_ref[g].astype(_BF16)

        logd_ref[g] = _vp(xb, dw1s[...], db1[...], dg1[...], dbe1[...],
                          dw2s[...], db2[...], dg2[...], dbe2[...],
                          dlw[...], dlb[...])

        durb = dur_ref[g].astype(_BF16)
        cum = jnp.dot(durb, tri, preferred_element_type=_F32)
        cumsh = jnp.concatenate([jnp.zeros((1, 1), _F32), cum[:, :-1]], axis=1)
        mlen_f = jnp.minimum(cum[:, S - 1:S], ml_ref[0, 0].astype(_F32))
        valid = pos < mlen_f
        oh = jnp.logical_and(cum > pos, cumsh <= pos)
        oh = jnp.logical_and(oh, valid).astype(_F32)
        x_exp = jnp.dot(oh, x_ref[g], preferred_element_type=_F32)

        xeb = x_exp.astype(_BF16)
        ppred_ref[g] = _vp(xeb, pw1s[...], pb1[...], pg1[...], pbe1[...],
                           pw2s[...], pb2[...], pg2[...], pbe2[...],
                           plw[...], plb[...])
        epred_ref[g] = _vp(xeb, ew1s[...], eb1[...], eg1[...], ebe1[...],
                           ew2s[...], eb2[...], eg2[...], ebe2[...],
                           elw[...], elb[...])

        ptc = pt_ref[g]
        etc = et_ref[g]
        ohp = ((pbh >= ptc) & (pbl < ptc)).astype(_F32)
        ohe = ((ebh >= etc) & (ebl < etc)).astype(_F32)
        pemb = jnp.dot(ohp, ptab_ref[...], preferred_element_type=_F32)
        eemb = jnp.dot(ohe, etab_ref[...], preferred_element_type=_F32)
        out_ref[g] = x_exp + pemb + eemb

        mel_i = jnp.minimum(cum[:, S - 1:S].astype(jnp.int32), ml_ref[0, 0])
        mellen_ref[g] = jnp.broadcast_to(mel_i, (1, 128))


def kernel(x, src_mask, mel_mask, duration_target, pitch_target, energy_target, max_len, pitch_bins, energy_bins, pitch_table, energy_table, dp_w1, dp_b1, dp_g1, dp_be1, dp_w2, dp_b2, dp_g2, dp_be2, dp_lw, dp_lb, pp_w1, pp_b1, pp_g1, pp_be1, pp_w2, pp_b2, pp_g2, pp_be2, pp_lw, pp_lb, ep_w1, ep_b1, ep_g1, ep_be1, ep_w2, ep_b2, ep_g2, ep_be2, ep_lw, ep_lb):
    B, S, H = x.shape
    T = mel_mask.shape[1]
    F = dp_b1.shape[0]
    NB = pitch_table.shape[0]
    NG = B // _G

    dur = duration_target.reshape(B, 1, S).astype(jnp.int32)
    pt = pitch_target.reshape(B, T, 1)
    et = energy_target.reshape(B, T, 1)
    ml = jnp.asarray(max_len, jnp.int32).reshape(1, 1)
    pbins = pitch_bins.reshape(1, NB - 1)
    ebins = energy_bins.reshape(1, NB - 1)

    def vp_args(w1, b1, g1, be1, w2, b2, g2, be2, lw, lb):
        return (w1, b1.reshape(1, F), g1.reshape(1, F), be1.reshape(1, F),
                w2, b2.reshape(1, F), g2.reshape(1, F), be2.reshape(1, F),
                lw, lb.reshape(1, 1))

    dp = vp_args(dp_w1, dp_b1, dp_g1, dp_be1, dp_w2, dp_b2, dp_g2, dp_be2, dp_lw, dp_lb)
    pp = vp_args(pp_w1, pp_b1, pp_g1, pp_be1, pp_w2, pp_b2, pp_g2, pp_be2, pp_lw, pp_lb)
    ep = vp_args(ep_w1, ep_b1, ep_g1, ep_be1, ep_w2, ep_b2, ep_g2, ep_be2, ep_lw, ep_lb)

    def full(a):
        return pl.BlockSpec(a.shape, lambda b: (0,) * a.ndim)

    in_specs = [
        pl.BlockSpec((_G, S, H), lambda b: (b, 0, 0)),
        pl.BlockSpec((_G, 1, S), lambda b: (b, 0, 0)),
        pl.BlockSpec((_G, T, 1), lambda b: (b, 0, 0)),
        pl.BlockSpec((_G, T, 1), lambda b: (b, 0, 0)),
        pl.BlockSpec(memory_space=pltpu.SMEM),
        full(pbins), full(ebins),
        full(pitch_table), full(energy_table),
    ]
    for grp in (dp, pp, ep):
        in_specs.extend(full(a) for a in grp)

    out_shapes = (
        jax.ShapeDtypeStruct((B, T, H), _F32),
        jax.ShapeDtypeStruct((B, S, 1), _F32),
        jax.ShapeDtypeStruct((B, T, 1), _F32),
        jax.ShapeDtypeStruct((B, T, 1), _F32),
        jax.ShapeDtypeStruct((B, 1, 128), jnp.int32),
    )
    out_specs = (
        pl.BlockSpec((_G, T, H), lambda b: (b, 0, 0)),
        pl.BlockSpec((_G, S, 1), lambda b: (b, 0, 0)),
        pl.BlockSpec((_G, T, 1), lambda b: (b, 0, 0)),
        pl.BlockSpec((_G, T, 1), lambda b: (b, 0, 0)),
        pl.BlockSpec((_G, 1, 128), lambda b: (b, 0, 0)),
    )

    out, logd, ppred, epred, mellen = pl.pallas_call(
        _body,
        grid=(NG,),
        in_specs=in_specs,
        out_specs=out_specs,
        out_shape=out_shapes,
        scratch_shapes=[
            pltpu.VMEM((3, H, F), _BF16), pltpu.VMEM((3, F, F), _BF16),
            pltpu.VMEM((3, H, F), _BF16), pltpu.VMEM((3, F, F), _BF16),
            pltpu.VMEM((3, H, F), _BF16), pltpu.VMEM((3, F, F), _BF16),
        ],
    )(x, dur, pt, et, ml, pbins, ebins,
      pitch_table, energy_table, *dp, *pp, *ep)

    logd2 = jnp.where(src_mask, 0.0, logd.reshape(B, S))
    ppred2 = jnp.where(mel_mask, 0.0, ppred.reshape(B, T))
    epred2 = jnp.where(mel_mask, 0.0, epred.reshape(B, T))
    return (out, logd2, ppred2, epred2, mellen[:, 0, 0], mel_mask)
